# baseline (speedup 1.0000x reference)
import functools

import jax
import jax.numpy as jnp
from jax import lax
from jax.experimental import pallas as pl
from jax.experimental.pallas import tpu as pltpu
from jax.experimental.pallas import tpu_sc as plsc

U_NUM = 50000
I_NUM = 10000
DIM = 128
E_NUM = 500000

NC = 2
NS = 16
NW = NC * NS
OP = 128
EW = 16384
EP = EW * NW
NOPS = EW // OP
IB = 8
NB = NOPS // IB
NR = 4
CG = 4
CW = DIM // CG
EROWS = EP // OP


def _make_spmm(n_out, cg):
  cw = DIM // cg
  nvr = cw // 16
  ZR = 200
  nch = n_out // ZR
  kmax = (nch + NS - 1) // NS

  def body(tbl, gidx4, sidx, val, out, acc, zbuf, gib, sib, vbuf, rows,
           g0, g1, msem):
    cid = lax.axis_index("c")
    sid = lax.axis_index("s")
    wid = sid * NC + cid
    ebase = wid * EW
    gsem = [g0, g1]

    @pl.loop(0, ZR)
    def _fill(e):
      for v in range(nvr):
        zbuf[e, pl.ds(16 * v, 16)] = jnp.zeros((16,), jnp.float32)

    def load_fire(op, slot, cgi):
      base = ebase + op * OP
      pltpu.sync_copy(gidx4.at[cgi, pl.ds(base, OP)], gib.at[slot])
      pltpu.sync_copy(sidx.at[pl.ds(base, OP)], sib.at[slot])
      pltpu.sync_copy(val.at[pl.ds(base, OP)], vbuf.at[slot])
      pltpu.async_copy(tbl.at[gib.at[slot]], rows.at[slot], gsem[slot])

    def wait_g(slot):
      pltpu.make_async_copy(tbl.at[gib.at[slot]], rows.at[slot],
                            gsem[slot]).wait()

    def scale(slot):
      @plsc.parallel_loop(0, OP // 16, unroll=2)
      def _s(g):
        vv = vbuf[slot, pl.ds(g * 16, 16)]
        for l in range(16):
          bb = lax.broadcast(vv[l], (16,))
          e = g * 16 + l
          for v in range(nvr):
            rows[slot, e, pl.ds(16 * v, 16)] = (
                rows[slot, e, pl.ds(16 * v, 16)] * bb)

    def scatter(slot):
      pltpu.sync_copy(rows.at[slot], acc.at[sib.at[slot]], add=True)

    @pl.loop(0, cg)
    def _pass(cgi):
      for k in range(kmax):
        ch = sid + NS * k

        @pl.when(ch < nch)
        def _z():
          pltpu.async_copy(zbuf, acc.at[pl.ds(ch * ZR, ZR)], msem)

      for k in range(kmax):
        ch = sid + NS * k

        @pl.when(ch < nch)
        def _zw():
          pltpu.make_async_copy(zbuf, acc.at[pl.ds(ch * ZR, ZR)], msem).wait()

      plsc.subcore_barrier()

      load_fire(0, 0, cgi)

      @pl.loop(0, NOPS // 2 - 1)
      def _t(t):
        load_fire(2 * t + 1, 1, cgi)
        wait_g(0)
        scale(0)
        scatter(0)
        load_fire(2 * t + 2, 0, cgi)
        wait_g(1)
        scale(1)
        scatter(1)

      load_fire(NOPS - 1, 1, cgi)
      wait_g(0)
      scale(0)
      scatter(0)
      wait_g(1)
      scale(1)
      scatter(1)
      plsc.subcore_barrier()

      for k in range(kmax):
        ch = sid + NS * k

        @pl.when(ch < nch)
        def _c():
          pltpu.async_copy(acc.at[pl.ds(ch * ZR, ZR)],
                           out.at[cid, cgi, pl.ds(ch * ZR, ZR)], msem)

      for k in range(kmax):
        ch = sid + NS * k

        @pl.when(ch < nch)
        def _cw():
          pltpu.make_async_copy(acc.at[pl.ds(ch * ZR, ZR)],
                                out.at[cid, cgi, pl.ds(ch * ZR, ZR)],
                                msem).wait()

      plsc.subcore_barrier()

  return pl.kernel(
      body,
      out_type=jax.ShapeDtypeStruct((NC, cg, n_out, cw), jnp.float32),
      mesh=plsc.VectorSubcoreMesh(core_axis_name="c", subcore_axis_name="s"),
      compiler_params=pltpu.CompilerParams(use_tc_tiling_on_sc=False),
      scratch_types=[
          pltpu.VMEM_SHARED((n_out, cw), jnp.float32),
          pltpu.VMEM((ZR, cw), jnp.float32),
          pltpu.VMEM((2, OP), jnp.int32),
          pltpu.VMEM((2, OP), jnp.int32),
          pltpu.VMEM((2, OP), jnp.float32),
          pltpu.VMEM((2, OP, cw), jnp.float32),
          pltpu.SemaphoreType.DMA, pltpu.SemaphoreType.DMA,
          pltpu.SemaphoreType.DMA,
      ],
  )


CGU = 4
CGI = 4
_spmm_u = _make_spmm(U_NUM, CGU)
_spmm_i = _make_spmm(I_NUM, CGI)


def _dense_body(tbl_ref, p4_ref, w0_ref, w1_ref, cw0_ref, cw1_ref, uep_ref,
                ue_ref, nxt_ref, mean_ref, *, first, cg):
  mparts = [(p4_ref[0, c].astype(jnp.float32)
             + p4_ref[1, c].astype(jnp.float32)) for c in range(cg)]
  msg = mparts[0] if cg == 1 else jnp.concatenate(mparts, axis=-1)
  tbl = tbl_ref[...]
  x = tbl + msg
  e0 = jax.nn.sigmoid(jnp.dot(x, w0_ref[...], preferred_element_type=jnp.float32))
  e1 = jax.nn.sigmoid(jnp.dot(x, w1_ref[...], preferred_element_type=jnp.float32))
  ue = (jnp.dot(e0, cw0_ref[...], preferred_element_type=jnp.float32)
        + jnp.dot(e1, cw1_ref[...], preferred_element_type=jnp.float32))
  ue_ref[...] = ue
  if first:
    nxt_ref[...] = tbl + ue
  else:
    mean_ref[...] = (ue + uep_ref[...]) * 0.5


def _dense(tbl, p4, w0, w1, cat_w, ue_prev, first, cg):
  n = tbl.shape[0]
  cw = DIM // cg
  bs = 1000
  grid = (n // bs,)
  cw0 = cat_w[:DIM]
  cw1 = cat_w[DIM:]
  row_spec = pl.BlockSpec((bs, DIM), lambda i: (i, 0))
  w_spec = pl.BlockSpec((DIM, DIM), lambda i: (0, 0))
  out_shapes = [jax.ShapeDtypeStruct((n, DIM), jnp.float32),
                jax.ShapeDtypeStruct((n, DIM), jnp.float32)]
  out_specs = [row_spec, row_spec]

  def kbody(tbl_ref, p4_ref, w0_ref, w1_ref, cw0_ref, cw1_ref, uep_ref, *outs):
    if first:
      ue_ref, nxt_ref = outs
      mean_ref = None
    else:
      ue_ref, mean_ref = outs
      nxt_ref = None
    _dense_body(tbl_ref, p4_ref, w0_ref, w1_ref, cw0_ref, cw1_ref, uep_ref,
                ue_ref, nxt_ref, mean_ref, first=first, cg=cg)

  return pl.pallas_call(
      kbody,
      grid=grid,
      in_specs=[
          row_spec,
          pl.BlockSpec((NC, cg, bs, cw), lambda i: (0, 0, i, 0)),
          w_spec, w_spec, w_spec, w_spec,
          row_spec,
      ],
      out_specs=out_specs,
      out_shape=out_shapes,
  )(tbl, p4, w0, w1, cw0, cw1, ue_prev)


def _repack(tbl, cg):
  n = tbl.shape[0]
  cw = DIM // cg
  t = tbl.reshape(n, cg, cw)
  if cg == 1:
    return t.reshape(n, cw)
  return t.transpose(1, 0, 2).reshape(cg * n, cw)


def _prep_edges(src, dst, val):
  pad = EP - E_NUM
  src = jnp.concatenate([src.astype(jnp.int32), jnp.zeros((pad,), jnp.int32)])
  dst = jnp.concatenate([dst.astype(jnp.int32), jnp.zeros((pad,), jnp.int32)])
  val = jnp.concatenate([val, jnp.zeros((pad,), jnp.float32)])
  gu = dst[None, :] + (jnp.arange(CGU, dtype=jnp.int32) * I_NUM)[:, None]
  gi = src[None, :] + (jnp.arange(CGI, dtype=jnp.int32) * U_NUM)[:, None]
  return (gu, src), (gi, dst), val


def kernel(user_table, item_table, u_w0, i_w0, u_w1, i_w1, u_cat_w, i_cat_w,
           edge_src_b0, edge_dst_b0, edge_val_b0,
           edge_src_b1, edge_dst_b1, edge_val_b1):
  pu0, pi0, v0 = _prep_edges(edge_src_b0, edge_dst_b0, edge_val_b0)
  pu1, pi1, v1 = _prep_edges(edge_src_b1, edge_dst_b1, edge_val_b1)

  i4 = _repack(item_table, CGU)
  u4 = _repack(user_table, CGI)

  up0 = _spmm_u(i4, pu0[0], pu0[1], v0)
  ip0 = _spmm_i(u4, pi0[0], pi0[1], v0)

  dummy_u = user_table
  dummy_i = item_table
  ue0, nu = _dense(user_table, up0, u_w0, u_w1, u_cat_w, dummy_u, True, CGU)
  ie0, ni = _dense(item_table, ip0, i_w0, i_w1, i_cat_w, dummy_i, True, CGI)

  up1 = _spmm_u(_repack(ni, CGU), pu1[0], pu1[1], v1)
  ip1 = _spmm_i(_repack(nu, CGI), pi1[0], pi1[1], v1)

  ue1, u_mean = _dense(nu, up1, u_w0, u_w1, u_cat_w, ue0, False, CGU)
  ie1, i_mean = _dense(ni, ip1, i_w0, i_w1, i_cat_w, ie0, False, CGI)

  return (u_mean, i_mean,
          jnp.stack([ue0, ue1], axis=0),
          jnp.stack([ie0, ie1], axis=0))


# speedup vs baseline: 1.2061x; 1.2061x over previous
import functools

import jax
import jax.numpy as jnp
from jax import lax
from jax.experimental import pallas as pl
from jax.experimental.pallas import tpu as pltpu
from jax.experimental.pallas import tpu_sc as plsc

U_NUM = 50000
I_NUM = 10000
DIM = 128
E_NUM = 500000

NC = 2
NS = 16
NW = NC * NS
OP = 128
EW = 16000
EP = EW * NW
NOPS = EW // OP
IB = 8
NB = NOPS // IB
NR = 4
CG = 4
CW = DIM // CG
EROWS = EP // OP


def _make_spmm(n_out, cg):
  cw = DIM // cg
  nvr = cw // 16
  ZR = 200
  nch = n_out // ZR
  kmax = (nch + NS - 1) // NS

  def body(tbl, gidx4, sidx, val, out, acc, zbuf, gib, sib, vbuf, rows,
           g0, g1):
    cid = lax.axis_index("c")
    sid = lax.axis_index("s")
    wid = sid * NC + cid
    ebase = wid * EW
    gsem = [g0, g1]

    @pl.loop(0, ZR)
    def _fill(e):
      for v in range(nvr):
        zbuf[e, pl.ds(16 * v, 16)] = jnp.zeros((16,), jnp.float32)

    def load_fire(op, slot, cgi):
      base = ebase + op * OP
      pltpu.sync_copy(gidx4.at[cgi, pl.ds(base, OP)], gib.at[slot])
      pltpu.sync_copy(sidx.at[pl.ds(base, OP)], sib.at[slot])
      pltpu.sync_copy(val.at[pl.ds(base, OP)], vbuf.at[pl.ds(slot * OP, OP)])
      pltpu.async_copy(tbl.at[gib.at[slot]], rows.at[slot], gsem[slot])

    def wait_g(slot):
      pltpu.make_async_copy(tbl.at[gib.at[slot]], rows.at[slot],
                            gsem[slot]).wait()

    def scale(slot):
      @plsc.parallel_loop(0, OP // 16, unroll=2)
      def _s(g):
        vv = vbuf[pl.ds(slot * OP + g * 16, 16)]
        for l in range(16):
          bb = lax.broadcast(vv[l], (16,))
          e = g * 16 + l
          for v in range(nvr):
            rows[slot, e, pl.ds(16 * v, 16)] = (
                rows[slot, e, pl.ds(16 * v, 16)] * bb)

    def scatter(slot):
      pltpu.sync_copy(rows.at[slot], acc.at[sib.at[slot]], add=True)

    @pl.loop(0, cg)
    def _pass(cgi):
      for k in range(kmax):
        ch = sid + NS * k

        @pl.when(ch < nch)
        def _z():
          pltpu.sync_copy(zbuf, acc.at[pl.ds(ch * ZR, ZR)])

      plsc.subcore_barrier()

      load_fire(0, 0, cgi)

      @pl.loop(0, (NOPS - 1) // 2)
      def _t(t):
        load_fire(2 * t + 1, 1, cgi)
        wait_g(0)
        scale(0)
        scatter(0)
        load_fire(2 * t + 2, 0, cgi)
        wait_g(1)
        scale(1)
        scatter(1)

      wait_g(0)
      scale(0)
      scatter(0)
      plsc.subcore_barrier()

      for k in range(kmax):
        ch = sid + NS * k

        @pl.when(ch < nch)
        def _c():
          pltpu.sync_copy(acc.at[pl.ds(ch * ZR, ZR)],
                          out.at[cid, cgi, pl.ds(ch * ZR, ZR)])

      plsc.subcore_barrier()

  return pl.kernel(
      body,
      out_type=jax.ShapeDtypeStruct((NC, cg, n_out, cw), jnp.float32),
      mesh=plsc.VectorSubcoreMesh(core_axis_name="c", subcore_axis_name="s"),
      compiler_params=pltpu.CompilerParams(use_tc_tiling_on_sc=False),
      scratch_types=[
          pltpu.VMEM_SHARED((n_out, cw), jnp.float32),
          pltpu.VMEM((ZR, cw), jnp.float32),
          pltpu.VMEM((2, OP), jnp.int32),
          pltpu.VMEM((2, OP), jnp.int32),
          pltpu.VMEM((2 * OP,), jnp.float32),
          pltpu.VMEM((2, OP, cw), jnp.float32),
          pltpu.SemaphoreType.DMA, pltpu.SemaphoreType.DMA,
      ],
  )


CGU = 4
CGI = 4
_spmm_u = _make_spmm(U_NUM, CGU)
_spmm_i = _make_spmm(I_NUM, CGI)


def _dense_body(tbl_ref, p4_ref, w0_ref, w1_ref, cw0_ref, cw1_ref, uep_ref,
                ue_ref, nxt_ref, mean_ref, *, first, cg):
  mparts = [(p4_ref[0, c].astype(jnp.float32)
             + p4_ref[1, c].astype(jnp.float32)) for c in range(cg)]
  msg = mparts[0] if cg == 1 else jnp.concatenate(mparts, axis=-1)
  tbl = tbl_ref[...]
  x = tbl + msg
  e0 = jax.nn.sigmoid(jnp.dot(x, w0_ref[...], preferred_element_type=jnp.float32))
  e1 = jax.nn.sigmoid(jnp.dot(x, w1_ref[...], preferred_element_type=jnp.float32))
  ue = (jnp.dot(e0, cw0_ref[...], preferred_element_type=jnp.float32)
        + jnp.dot(e1, cw1_ref[...], preferred_element_type=jnp.float32))
  ue_ref[...] = ue
  if first:
    nxt_ref[...] = tbl + ue
  else:
    mean_ref[...] = (ue + uep_ref[...]) * 0.5


def _dense(tbl, p4, w0, w1, cat_w, ue_prev, first, cg):
  n = tbl.shape[0]
  cw = DIM // cg
  bs = 1000
  grid = (n // bs,)
  cw0 = cat_w[:DIM]
  cw1 = cat_w[DIM:]
  row_spec = pl.BlockSpec((bs, DIM), lambda i: (i, 0))
  w_spec = pl.BlockSpec((DIM, DIM), lambda i: (0, 0))
  out_shapes = [jax.ShapeDtypeStruct((n, DIM), jnp.float32),
                jax.ShapeDtypeStruct((n, DIM), jnp.float32)]
  out_specs = [row_spec, row_spec]

  def kbody(tbl_ref, p4_ref, w0_ref, w1_ref, cw0_ref, cw1_ref, uep_ref, *outs):
    if first:
      ue_ref, nxt_ref = outs
      mean_ref = None
    else:
      ue_ref, mean_ref = outs
      nxt_ref = None
    _dense_body(tbl_ref, p4_ref, w0_ref, w1_ref, cw0_ref, cw1_ref, uep_ref,
                ue_ref, nxt_ref, mean_ref, first=first, cg=cg)

  return pl.pallas_call(
      kbody,
      grid=grid,
      in_specs=[
          row_spec,
          pl.BlockSpec((NC, cg, bs, cw), lambda i: (0, 0, i, 0)),
          w_spec, w_spec, w_spec, w_spec,
          row_spec,
      ],
      out_specs=out_specs,
      out_shape=out_shapes,
  )(tbl, p4, w0, w1, cw0, cw1, ue_prev)


def _repack(tbl, cg):
  n = tbl.shape[0]
  cw = DIM // cg
  t = tbl.reshape(n, cg, cw)
  if cg == 1:
    return t.reshape(n, cw)
  return t.transpose(1, 0, 2).reshape(cg * n, cw)


def _prep_edges(src, dst, val):
  pad = EP - E_NUM
  src = jnp.concatenate([src.astype(jnp.int32), jnp.zeros((pad,), jnp.int32)])
  dst = jnp.concatenate([dst.astype(jnp.int32), jnp.zeros((pad,), jnp.int32)])
  val = jnp.concatenate([val, jnp.zeros((pad,), jnp.float32)])
  gu = dst[None, :] + (jnp.arange(CGU, dtype=jnp.int32) * I_NUM)[:, None]
  gi = src[None, :] + (jnp.arange(CGI, dtype=jnp.int32) * U_NUM)[:, None]
  return (gu, src), (gi, dst), val


def kernel(user_table, item_table, u_w0, i_w0, u_w1, i_w1, u_cat_w, i_cat_w,
           edge_src_b0, edge_dst_b0, edge_val_b0,
           edge_src_b1, edge_dst_b1, edge_val_b1):
  pu0, pi0, v0 = _prep_edges(edge_src_b0, edge_dst_b0, edge_val_b0)
  pu1, pi1, v1 = _prep_edges(edge_src_b1, edge_dst_b1, edge_val_b1)

  i4 = _repack(item_table, CGU)
  u4 = _repack(user_table, CGI)

  up0 = _spmm_u(i4, pu0[0], pu0[1], v0)
  ip0 = _spmm_i(u4, pi0[0], pi0[1], v0)

  dummy_u = user_table
  dummy_i = item_table
  ue0, nu = _dense(user_table, up0, u_w0, u_w1, u_cat_w, dummy_u, True, CGU)
  ie0, ni = _dense(item_table, ip0, i_w0, i_w1, i_cat_w, dummy_i, True, CGI)

  up1 = _spmm_u(_repack(ni, CGU), pu1[0], pu1[1], v1)
  ip1 = _spmm_i(_repack(nu, CGI), pi1[0], pi1[1], v1)

  ue1, u_mean = _dense(nu, up1, u_w0, u_w1, u_cat_w, ue0, False, CGU)
  ie1, i_mean = _dense(ni, ip1, i_w0, i_w1, i_cat_w, ie0, False, CGI)

  return (u_mean, i_mean,
          jnp.stack([ue0, ue1], axis=0),
          jnp.stack([ie0, ie1], axis=0))


# speedup vs baseline: 1.2626x; 1.0468x over previous
import functools

import jax
import jax.numpy as jnp
from jax import lax
from jax.experimental import pallas as pl
from jax.experimental.pallas import tpu as pltpu
from jax.experimental.pallas import tpu_sc as plsc

U_NUM = 50000
I_NUM = 10000
DIM = 128
E_NUM = 500000

NC = 2
NS = 16
NW = NC * NS
OP = 128
EW = 16000
EP = EW * NW
NOPS = EW // OP
IB = 8
NB = NOPS // IB
NR = 4
CG = 4
CW = DIM // CG
EROWS = EP // OP


def _make_spmm(n_out, cg):
  cw = DIM // cg
  nvr = cw // 16
  ZR = 200
  nch = n_out // ZR
  kmax = (nch + NS - 1) // NS

  def body(tbl, gidx4, sidx, val, out, acc, zbuf, gib, sib, vbuf, rows,
           g0, g1):
    cid = lax.axis_index("c")
    sid = lax.axis_index("s")
    wid = sid * NC + cid
    ebase = wid * EW
    gsem = [g0, g1]

    @pl.loop(0, ZR)
    def _fill(e):
      for v in range(nvr):
        zbuf[e, pl.ds(16 * v, 16)] = jnp.zeros((16,), jnp.float32)

    def load_fire(op, slot, cgi):
      base = ebase + op * OP
      pltpu.sync_copy(gidx4.at[cgi, pl.ds(base, OP)], gib.at[slot])
      pltpu.sync_copy(sidx.at[pl.ds(base, OP)], sib.at[slot])
      pltpu.sync_copy(val.at[pl.ds(base, OP)], vbuf.at[pl.ds(slot * OP, OP)])
      pltpu.async_copy(tbl.at[gib.at[slot]], rows.at[slot], gsem[slot])

    def wait_g(slot):
      pltpu.make_async_copy(tbl.at[gib.at[slot]], rows.at[slot],
                            gsem[slot]).wait()

    def scale(slot):
      @plsc.parallel_loop(0, OP // 16, unroll=2)
      def _s(g):
        vv = vbuf[pl.ds(slot * OP + g * 16, 16)]
        for l in range(16):
          bb = lax.broadcast(vv[l], (16,))
          e = g * 16 + l
          for v in range(nvr):
            rows[slot, e, pl.ds(16 * v, 16)] = (
                rows[slot, e, pl.ds(16 * v, 16)] * bb)

    def scatter(slot):
      pltpu.sync_copy(rows.at[slot], acc.at[sib.at[slot]], add=True)

    @pl.loop(0, cg)
    def _pass(cgi):
      for k in range(kmax):
        ch = sid + NS * k

        @pl.when(ch < nch)
        def _z():
          pltpu.sync_copy(zbuf, acc.at[pl.ds(ch * ZR, ZR)])

      plsc.subcore_barrier()

      load_fire(0, 0, cgi)

      @pl.loop(0, (NOPS - 1) // 2)
      def _t(t):
        load_fire(2 * t + 1, 1, cgi)
        wait_g(0)
        scale(0)
        scatter(0)
        load_fire(2 * t + 2, 0, cgi)
        wait_g(1)
        scale(1)
        scatter(1)

      wait_g(0)
      scale(0)
      scatter(0)
      plsc.subcore_barrier()

      for k in range(kmax):
        ch = sid + NS * k

        @pl.when(ch < nch)
        def _c():
          pltpu.sync_copy(acc.at[pl.ds(ch * ZR, ZR)],
                          out.at[cid, cgi, pl.ds(ch * ZR, ZR)])

      plsc.subcore_barrier()

  return pl.kernel(
      body,
      out_type=jax.ShapeDtypeStruct((NC, cg, n_out, cw), jnp.float32),
      mesh=plsc.VectorSubcoreMesh(core_axis_name="c", subcore_axis_name="s"),
      compiler_params=pltpu.CompilerParams(use_tc_tiling_on_sc=False),
      scratch_types=[
          pltpu.VMEM_SHARED((n_out, cw), jnp.float32),
          pltpu.VMEM((ZR, cw), jnp.float32),
          pltpu.VMEM((2, OP), jnp.int32),
          pltpu.VMEM((2, OP), jnp.int32),
          pltpu.VMEM((2 * OP,), jnp.float32),
          pltpu.VMEM((2, OP, cw), jnp.float32),
          pltpu.SemaphoreType.DMA, pltpu.SemaphoreType.DMA,
      ],
  )


CGU = 4
CGI = 4
_spmm_u = _make_spmm(U_NUM, CGU)
_spmm_i = _make_spmm(I_NUM, CGI)


def _dense_body(tbl_ref, p4_ref, w0_ref, w1_ref, cw0_ref, cw1_ref, uep_ref,
                ue_ref, nxt_ref, nxt4_ref, mean_ref, *, first, cg):
  mparts = [(p4_ref[0, c].astype(jnp.float32)
             + p4_ref[1, c].astype(jnp.float32)) for c in range(cg)]
  msg = mparts[0] if cg == 1 else jnp.concatenate(mparts, axis=-1)
  tbl = tbl_ref[...]
  x = tbl + msg
  e0 = jax.nn.sigmoid(jnp.dot(x, w0_ref[...], preferred_element_type=jnp.float32))
  e1 = jax.nn.sigmoid(jnp.dot(x, w1_ref[...], preferred_element_type=jnp.float32))
  ue = (jnp.dot(e0, cw0_ref[...], preferred_element_type=jnp.float32)
        + jnp.dot(e1, cw1_ref[...], preferred_element_type=jnp.float32))
  ue_ref[...] = ue
  if first:
    nxt = tbl + ue
    nxt_ref[...] = nxt
    cw = DIM // cg
    for c in range(cg):
      nxt4_ref[c] = nxt[:, c * cw:(c + 1) * cw]
  else:
    mean_ref[...] = (ue + uep_ref[...]) * 0.5


def _dense(tbl, p4, w0, w1, cat_w, ue_prev, first, cg):
  n = tbl.shape[0]
  cw = DIM // cg
  bs = 1000
  grid = (n // bs,)
  cw0 = cat_w[:DIM]
  cw1 = cat_w[DIM:]
  row_spec = pl.BlockSpec((bs, DIM), lambda i: (i, 0))
  w_spec = pl.BlockSpec((DIM, DIM), lambda i: (0, 0))
  out_shapes = [jax.ShapeDtypeStruct((n, DIM), jnp.float32),
                jax.ShapeDtypeStruct((n, DIM), jnp.float32)]
  out_specs = [row_spec, row_spec]
  if first:
    out_shapes[1:] = [jax.ShapeDtypeStruct((n, DIM), jnp.float32),
                      jax.ShapeDtypeStruct((cg, n, cw), jnp.float32)]
    out_specs[1:] = [row_spec, pl.BlockSpec((cg, bs, cw), lambda i: (0, i, 0))]

  def kbody(tbl_ref, p4_ref, w0_ref, w1_ref, cw0_ref, cw1_ref, uep_ref, *outs):
    if first:
      ue_ref, nxt_ref, nxt4_ref = outs
      mean_ref = None
    else:
      ue_ref, mean_ref = outs
      nxt_ref = nxt4_ref = None
    _dense_body(tbl_ref, p4_ref, w0_ref, w1_ref, cw0_ref, cw1_ref, uep_ref,
                ue_ref, nxt_ref, nxt4_ref, mean_ref, first=first, cg=cg)

  return pl.pallas_call(
      kbody,
      grid=grid,
      in_specs=[
          row_spec,
          pl.BlockSpec((NC, cg, bs, cw), lambda i: (0, 0, i, 0)),
          w_spec, w_spec, w_spec, w_spec,
          row_spec,
      ],
      out_specs=out_specs,
      out_shape=out_shapes,
  )(tbl, p4, w0, w1, cw0, cw1, ue_prev)


def _repack(tbl, cg):
  n = tbl.shape[0]
  cw = DIM // cg
  t = tbl.reshape(n, cg, cw)
  if cg == 1:
    return t.reshape(n, cw)
  return t.transpose(1, 0, 2).reshape(cg * n, cw)


def _prep_edges(src, dst, val):
  pad = EP - E_NUM
  src = jnp.concatenate([src.astype(jnp.int32), jnp.zeros((pad,), jnp.int32)])
  dst = jnp.concatenate([dst.astype(jnp.int32), jnp.zeros((pad,), jnp.int32)])
  val = jnp.concatenate([val, jnp.zeros((pad,), jnp.float32)])
  gu = dst[None, :] + (jnp.arange(CGU, dtype=jnp.int32) * I_NUM)[:, None]
  gi = src[None, :] + (jnp.arange(CGI, dtype=jnp.int32) * U_NUM)[:, None]
  return (gu, src), (gi, dst), val


def kernel(user_table, item_table, u_w0, i_w0, u_w1, i_w1, u_cat_w, i_cat_w,
           edge_src_b0, edge_dst_b0, edge_val_b0,
           edge_src_b1, edge_dst_b1, edge_val_b1):
  pu0, pi0, v0 = _prep_edges(edge_src_b0, edge_dst_b0, edge_val_b0)
  pu1, pi1, v1 = _prep_edges(edge_src_b1, edge_dst_b1, edge_val_b1)

  i4 = _repack(item_table, CGU)
  u4 = _repack(user_table, CGI)

  up0 = _spmm_u(i4, pu0[0], pu0[1], v0)
  ip0 = _spmm_i(u4, pi0[0], pi0[1], v0)

  dummy_u = user_table
  dummy_i = item_table
  ue0, nu, nu4 = _dense(user_table, up0, u_w0, u_w1, u_cat_w, dummy_u, True,
                        CGI)
  ie0, ni, ni4 = _dense(item_table, ip0, i_w0, i_w1, i_cat_w, dummy_i, True,
                        CGU)

  up1 = _spmm_u(ni4.reshape(CGU * I_NUM, DIM // CGU), pu1[0], pu1[1], v1)
  ip1 = _spmm_i(nu4.reshape(CGI * U_NUM, DIM // CGI), pi1[0], pi1[1], v1)

  ue1, u_mean = _dense(nu, up1, u_w0, u_w1, u_cat_w, ue0, False, CGU)
  ie1, i_mean = _dense(ni, ip1, i_w0, i_w1, i_cat_w, ie0, False, CGI)

  return (u_mean, i_mean,
          jnp.stack([ue0, ue1], axis=0),
          jnp.stack([ie0, ie1], axis=0))
